# baseline (speedup 1.0000x reference)
import functools

import jax
import jax.numpy as jnp
from jax import lax
from jax.experimental import pallas as pl
from jax.experimental.pallas import tpu as pltpu
from jax.experimental.pallas import tpu_sc as plsc

G = 2
NV = 320
D = 1024
VD = 512

BLK = 512


def _mlp_argmax_body(x_ref, w1_ref, b1_ref, w2_ref, b2_ref, idx_ref):
    x = x_ref[...]
    h = x @ w1_ref[...] + b1_ref[...]
    h = h * 0.5 * (1.0 + lax.erf(h * jnp.float32(0.7071067811865476)))
    logits = h @ w2_ref[...] + b2_ref[...]
    col = lax.broadcasted_iota(jnp.int32, (BLK, G * NV), 1)
    neg = jnp.float32(-jnp.inf)
    big = jnp.int32(2**30)
    outs = []
    for g in range(G):
        mask = (col >= g * NV) & (col < (g + 1) * NV)
        m = jnp.max(jnp.where(mask, logits, neg), axis=1, keepdims=True)
        hit = (logits == m) & mask
        outs.append(jnp.min(jnp.where(hit, col, big), axis=1))
    idx_ref[0] = jnp.stack(outs)


def _tc_mlp_argmax(xf, w1t, b1, w2t, b2):
    n = xf.shape[0]
    nblk = n // BLK
    return pl.pallas_call(
        _mlp_argmax_body,
        grid=(nblk,),
        in_specs=[
            pl.BlockSpec((BLK, D), lambda i: (i, 0)),
            pl.BlockSpec((D, D), lambda i: (0, 0)),
            pl.BlockSpec((1, D), lambda i: (0, 0)),
            pl.BlockSpec((D, G * NV), lambda i: (0, 0)),
            pl.BlockSpec((1, G * NV), lambda i: (0, 0)),
        ],
        out_specs=pl.BlockSpec((1, G, BLK), lambda i: (i, 0, 0)),
        out_shape=jax.ShapeDtypeStruct((nblk, G, BLK), jnp.int32),
    )(xf, w1t, b1, w2t, b2)


def _make_sc_gather(n_rows):
    info = plsc.get_sparse_core_info()
    nw = info.num_cores * info.num_subcores
    rows_per_w = n_rows // nw
    chunk = 64
    nchunks = rows_per_w // chunk
    mesh = plsc.VectorSubcoreMesh(core_axis_name="c", subcore_axis_name="s")

    @functools.partial(
        pl.kernel,
        mesh=mesh,
        out_type=jax.ShapeDtypeStruct((n_rows, VD), jnp.float32),
        scratch_types=[
            pltpu.VMEM((rows_per_w,), jnp.int32),
            pltpu.VMEM((2, chunk, VD), jnp.float32),
            pltpu.SemaphoreType.DMA,
            pltpu.SemaphoreType.DMA,
        ],
    )
    def gather(table_hbm, idx_hbm, out_hbm, idx_v, rows_v, gsem0, gsem1):
        wid = lax.axis_index("s") * info.num_cores + lax.axis_index("c")
        base = wid * rows_per_w
        pltpu.sync_copy(idx_hbm.at[pl.ds(base, rows_per_w)], idx_v)
        gsems = (gsem0, gsem1)
        cp0 = pltpu.async_copy(
            table_hbm.at[idx_v.at[pl.ds(0, chunk)]], rows_v.at[0], gsems[0]
        )
        cps = [cp0, None]
        for c in range(nchunks):
            b = c % 2
            cps[b].wait()
            if c + 1 < nchunks:
                nb = (c + 1) % 2
                cps[nb] = pltpu.async_copy(
                    table_hbm.at[idx_v.at[pl.ds((c + 1) * chunk, chunk)]],
                    rows_v.at[nb],
                    gsems[nb],
                )
            pltpu.sync_copy(rows_v.at[b], out_hbm.at[pl.ds(base + c * chunk, chunk)])

    return gather


NSLICE = 4


def kernel(x, W1, b1, W2, b2, codebook):
    bsz, tsz, fsz = x.shape
    xf = x.reshape(-1, fsz)
    n = xf.shape[0]
    w1t, w2t = W1.T, W2.T
    b1r, b2r = b1.reshape(1, D), b2.reshape(1, G * NV)
    table = codebook.reshape(G * NV, VD)
    ns = n // NSLICE
    sc_gather = _make_sc_gather(ns * G)
    outs = []
    for s in range(NSLICE):
        idx = _tc_mlp_argmax(xf[s * ns:(s + 1) * ns], w1t, b1r, w2t, b2r)
        idx_flat = idx.transpose(0, 2, 1).reshape(-1)
        outs.append(sc_gather(table, idx_flat))
    return jnp.concatenate(outs).reshape(bsz, tsz, G * VD)


# speedup vs baseline: 1.6954x; 1.6954x over previous
import functools

import jax
import jax.numpy as jnp
from jax import lax
from jax.experimental import pallas as pl
from jax.experimental.pallas import tpu as pltpu
from jax.experimental.pallas import tpu_sc as plsc

G = 2
NV = 320
D = 1024
VD = 512

BLK = 512


def _mlp_argmax_body(x_ref, w1_ref, b1_ref, w2_ref, b2_ref, idx_ref):
    x = x_ref[...]
    h = lax.dot_general(x, w1_ref[...], (((1,), (1,)), ((), ())),
                        preferred_element_type=jnp.float32)
    h = h + b1_ref[...]
    h = h * 0.5 * (1.0 + lax.erf(h * jnp.float32(0.7071067811865476)))
    logits = lax.dot_general(h, w2_ref[...], (((1,), (1,)), ((), ())),
                             preferred_element_type=jnp.float32)
    logits = logits + b2_ref[...]
    col = lax.broadcasted_iota(jnp.int32, (BLK, G * NV), 1)
    neg = jnp.float32(-jnp.inf)
    big = jnp.int32(2**30)
    outs = []
    for g in range(G):
        mask = (col >= g * NV) & (col < (g + 1) * NV)
        m = jnp.max(jnp.where(mask, logits, neg), axis=1, keepdims=True)
        hit = (logits == m) & mask
        outs.append(jnp.min(jnp.where(hit, col, big), axis=1))
    idx_ref[0] = jnp.stack(outs)


def _tc_mlp_argmax(xf, w1, b1, w2, b2):
    n = xf.shape[0]
    nblk = n // BLK
    return pl.pallas_call(
        _mlp_argmax_body,
        grid=(nblk,),
        in_specs=[
            pl.BlockSpec((BLK, D), lambda i: (i, 0)),
            pl.BlockSpec((D, D), lambda i: (0, 0)),
            pl.BlockSpec((1, D), lambda i: (0, 0)),
            pl.BlockSpec((G * NV, D), lambda i: (0, 0)),
            pl.BlockSpec((1, G * NV), lambda i: (0, 0)),
        ],
        out_specs=pl.BlockSpec((1, G, BLK), lambda i: (i, 0, 0)),
        out_shape=jax.ShapeDtypeStruct((nblk, G, BLK), jnp.int32),
    )(xf, w1, b1, w2, b2)


def _make_sc_gather(n_tok):
    info = plsc.get_sparse_core_info()
    nw = info.num_cores * info.num_subcores
    tok_per_w = n_tok // nw
    chunk = 64
    ntch = tok_per_w // chunk
    nchunks = ntch * G
    mesh = plsc.VectorSubcoreMesh(core_axis_name="c", subcore_axis_name="s")

    @functools.partial(
        pl.kernel,
        mesh=mesh,
        out_type=jax.ShapeDtypeStruct((n_tok, G * VD), jnp.float32),
        scratch_types=[
            pltpu.VMEM((G, tok_per_w), jnp.int32),
            pltpu.VMEM((2, chunk, VD), jnp.float32),
            pltpu.SemaphoreType.DMA,
            pltpu.SemaphoreType.DMA,
        ],
    )
    def gather(table_hbm, idx_hbm, out_hbm, idx_v, rows_v, gsem0, gsem1):
        wid = lax.axis_index("s") * info.num_cores + lax.axis_index("c")
        t0 = wid * tok_per_w
        i_blk = t0 // BLK
        off = t0 % BLK
        pltpu.sync_copy(idx_hbm.at[i_blk, :, pl.ds(off, tok_per_w)], idx_v)
        gsems = (gsem0, gsem1)

        def start(c, b):
            j, g = c // G, c % G
            return pltpu.async_copy(
                table_hbm.at[idx_v.at[g, pl.ds(j * chunk, chunk)]],
                rows_v.at[b],
                gsems[b],
            )

        cps = [start(0, 0), None]
        for c in range(nchunks):
            b = c % 2
            j, g = c // G, c % G
            cps[b].wait()
            if c + 1 < nchunks:
                cps[1 - b] = start(c + 1, 1 - b)
            pltpu.sync_copy(
                rows_v.at[b],
                out_hbm.at[pl.ds(t0 + j * chunk, chunk), pl.ds(g * VD, VD)],
            )

    return gather


def kernel(x, W1, b1, W2, b2, codebook):
    bsz, tsz, fsz = x.shape
    xf = x.reshape(-1, fsz)
    n = xf.shape[0]
    idx = _tc_mlp_argmax(xf, W1, b1.reshape(1, D), W2, b2.reshape(1, G * NV))
    table = codebook.reshape(G * NV, VD)
    rows = _make_sc_gather(n)(table, idx)
    return rows.reshape(bsz, tsz, G * VD)
